# baseline (speedup 1.0000x reference)
import functools

import jax
import jax.numpy as jnp
from jax import lax
from jax.experimental import pallas as pl
from jax.experimental.pallas import tpu as pltpu
from jax.experimental.pallas import tpu_sc as plsc

B, T, DXN, RXN = 128, 20, 40, 40
D = 256
DXV, RXV = 4880, 3000
ATTN = 128
DPL = 4880
SEG = B * T

NC, NS = 2, 16
NW = NC * NS
SEGW = SEG // NW


_NBUF = 4


def _make_bag(nseg):
  segw = nseg // NW
  nchunk = segw
  mesh = plsc.VectorSubcoreMesh(core_axis_name="c", subcore_axis_name="s")

  @functools.partial(
      pl.kernel,
      mesh=mesh,
      out_type=jax.ShapeDtypeStruct((nseg, D), jnp.float32),
      scratch_types=[
          pltpu.VMEM((segw * DXN,), jnp.int32),
          pltpu.VMEM((segw * RXN,), jnp.int32),
          pltpu.VMEM((_NBUF, DXN, D), jnp.float32),
          pltpu.VMEM((_NBUF, RXN, D), jnp.float32),
          pltpu.VMEM((segw, D), jnp.float32),
      ] + [pltpu.SemaphoreType.DMA] * (2 * _NBUF),
  )
  def bag(dxi_hbm, rxi_hbm, dxemb_hbm, rxemb_hbm, out_hbm,
          dxi_v, rxi_v, rows_dx, rows_rx, acc_v, *sems_flat):
    wid = lax.axis_index("s") * NC + lax.axis_index("c")
    base = wid * segw
    pltpu.sync_copy(dxi_hbm.at[pl.ds(base * DXN, segw * DXN)], dxi_v)
    pltpu.sync_copy(rxi_hbm.at[pl.ds(base * RXN, segw * RXN)], rxi_v)
    sems = tuple(
        (sems_flat[2 * u], sems_flat[2 * u + 1]) for u in range(_NBUF))

    def issue(c, slot):
      sd, sr = sems[slot]
      pltpu.async_copy(
          dxemb_hbm.at[dxi_v.at[pl.ds(c * DXN, DXN)]], rows_dx.at[slot], sd)
      pltpu.async_copy(
          rxemb_hbm.at[rxi_v.at[pl.ds(c * RXN, RXN)]], rows_rx.at[slot], sr)

    def consume(c, slot):
      def row_body(h, acc):
        r = 2 * h
        return tuple(
            acc[j]
            + (rows_dx[slot, r, pl.ds(16 * j, 16)]
               + rows_rx[slot, r, pl.ds(16 * j, 16)])
            + (rows_dx[slot, r + 1, pl.ds(16 * j, 16)]
               + rows_rx[slot, r + 1, pl.ds(16 * j, 16)])
            for j in range(D // 16))

      zeros = tuple(jnp.zeros((16,), jnp.float32) for _ in range(D // 16))
      acc = lax.fori_loop(0, DXN // 2, row_body, zeros)
      for j in range(D // 16):
        acc_v[c, pl.ds(16 * j, 16)] = acc[j]

    def drain(slot):
      sd, sr = sems[slot]
      pltpu.make_async_copy(
          dxemb_hbm.at[dxi_v.at[pl.ds(0, DXN)]], rows_dx.at[slot], sd).wait()
      pltpu.make_async_copy(
          rxemb_hbm.at[rxi_v.at[pl.ds(0, RXN)]], rows_rx.at[slot], sr).wait()

    for u in range(_NBUF - 1):
      issue(u, u)

    def ring_body(i, carry):
      c0 = _NBUF * i
      for u in range(_NBUF):
        c = c0 + u
        drain(u)
        consume(c, u)
        nxt = c + _NBUF - 1

        @pl.when(nxt < nchunk)
        def _():
          issue(nxt, (u + _NBUF - 1) % _NBUF)

      return carry

    lax.fori_loop(0, nchunk // _NBUF, ring_body, 0)
    pltpu.sync_copy(acc_v, out_hbm.at[pl.ds(base, segw)])

  return bag


_HALVES = 2
_BH = B // _HALVES
_SEGH = _BH * T
_bag = _make_bag(_SEGH)


def _attn_body(nb, pooled_ref, attn_w_ref, attn_b_ref, attnc_w_ref,
               attnc_b_ref, dp_w_ref, dp_b_ref, read_w_ref, read_b_ref,
               mort_w_ref, mort_b_ref, visit_out, dp_out, read_out, mort_out):
  visit = jnp.tanh(pooled_ref[...])
  v3 = visit.reshape(nb, T, D)
  last = v3[:, T - 1, :]
  w1 = attn_w_ref[0:D, :]
  w2 = attn_w_ref[D:2 * D, :]
  h = jnp.dot(visit, w1, preferred_element_type=jnp.float32)
  h2 = jnp.dot(last, w2, preferred_element_type=jnp.float32)
  e = jnp.tanh(h.reshape(nb, T, ATTN) + h2[:, None, :] + attn_b_ref[...])
  sc = jnp.sum(e * attnc_w_ref[...][None, :, :], axis=-1) + attnc_b_ref[0, 0]
  m = jnp.max(sc, axis=1, keepdims=True)
  a = jnp.exp(sc - m)
  alpha = a / jnp.sum(a, axis=1, keepdims=True)
  pt = jnp.sum(alpha[:, :, None] * v3, axis=1)
  dp = jax.nn.sigmoid(
      jnp.dot(pt, dp_w_ref[...], preferred_element_type=jnp.float32)
      + dp_b_ref[...])
  rd = jax.nn.sigmoid(
      jnp.sum(pt * read_w_ref[...], axis=-1, keepdims=True) + read_b_ref[0, 0])
  mt = jax.nn.sigmoid(
      jnp.sum(pt * mort_w_ref[...], axis=-1, keepdims=True) + mort_b_ref[0, 0])
  visit_out[...] = visit.astype(jnp.bfloat16)
  dp_out[...] = dp
  read_out[...] = rd
  mort_out[...] = mt


def _attn_call(nb, pooled, attn_w, attn_b, attnc_w, attnc_b, dp_w, dp_b,
               read_w, read_b, mort_w, mort_b):
  nseg = nb * T
  return pl.pallas_call(
      functools.partial(_attn_body, nb),
      out_shape=(
          jax.ShapeDtypeStruct((nseg, D), jnp.bfloat16),
          jax.ShapeDtypeStruct((nb, DPL), jnp.float32),
          jax.ShapeDtypeStruct((nb, 1), jnp.float32),
          jax.ShapeDtypeStruct((nb, 1), jnp.float32),
      ),
  )(pooled, attn_w, attn_b, attnc_w, attnc_b, dp_w, dp_b, read_w, read_b,
    mort_w, mort_b)


_VT = 512


def _loss_body(nb, vocab, visit_ref, w_ref, b_ref, lab_ref,
               z_out, s1_out, sl_out):
  nseg = nb * T
  i = pl.program_id(0)

  @pl.when(i == 0)
  def _():
    z_out[...] = jnp.zeros_like(z_out)
    s1_out[...] = jnp.zeros_like(s1_out)
    sl_out[...] = jnp.zeros_like(sl_out)

  logits = (jnp.dot(visit_ref[...], w_ref[...],
                    preferred_element_type=jnp.float32) + b_ref[...])
  col = i * _VT + lax.broadcasted_iota(jnp.int32, (1, _VT), 1)
  valid = col < vocab
  lab = lab_ref[...].reshape(nseg, _VT)
  expl = jnp.where(valid, jnp.exp(logits), 0.0)
  s1 = jnp.where(valid, lab * logits, 0.0)
  sl = jnp.where(valid, lab, 0.0)
  z_out[...] += jnp.sum(expl, axis=1, keepdims=True)
  s1_out[...] += jnp.sum(s1, axis=1, keepdims=True)
  sl_out[...] += jnp.sum(sl, axis=1, keepdims=True)


def _loss_call(nb, visit_bf, w_bf, b2, labels, vocab):
  nseg = nb * T
  num_tiles = pl.cdiv(vocab, _VT)
  one = jax.ShapeDtypeStruct((nseg, 1), jnp.float32)
  return pl.pallas_call(
      functools.partial(_loss_body, nb, vocab),
      grid=(num_tiles,),
      in_specs=[
          pl.BlockSpec((nseg, D), lambda i: (0, 0)),
          pl.BlockSpec((D, _VT), lambda i: (0, i)),
          pl.BlockSpec((1, _VT), lambda i: (0, i)),
          pl.BlockSpec((nb, T, _VT), lambda i: (0, 0, i)),
      ],
      out_specs=(
          pl.BlockSpec((nseg, 1), lambda i: (0, 0)),
          pl.BlockSpec((nseg, 1), lambda i: (0, 0)),
          pl.BlockSpec((nseg, 1), lambda i: (0, 0)),
      ),
      out_shape=(one, one, one),
  )(visit_bf, w_bf, b2, labels)


def _combine_body(*refs):
  parts, out_ref = refs[:-1], refs[-1]
  total = None
  for h in range(_HALVES):
    z1, s11, sl1, z2, s12, sl2 = parts[6 * h:6 * h + 6]
    z = z1[...] + z2[...]
    s1 = s11[...] + s12[...]
    sl = sl1[...] + sl2[...]
    rows = s1 - jnp.log(z) * sl
    t = jnp.sum(rows, axis=0, keepdims=True)
    total = t if total is None else total + t
  out_ref[...] = -total / B


def _combine_call(parts):
  return pl.pallas_call(
      _combine_body,
      out_shape=jax.ShapeDtypeStruct((1, 1), jnp.float32),
  )(*parts)


def kernel(dxseqs, drugseqs, dx_onehot, drug_onehot, EHRdxEmb, EHRdrugEmb,
           attn_W, attn_b, attnC_W, attnC_b, dp_W, dp_b, read_W, read_b,
           mort_W, mort_b, co_W, co_b):
  co_Wb = co_W.astype(jnp.bfloat16)
  a_b = attn_b.reshape(1, ATTN)
  ac_w = attnC_W.reshape(1, ATTN)
  ac_b = attnC_b.reshape(1, 1)
  dp_b2 = dp_b.reshape(1, DPL)
  r_w = read_W.reshape(1, D)
  r_b = read_b.reshape(1, 1)
  m_w = mort_W.reshape(1, D)
  m_b = mort_b.reshape(1, 1)
  cb_dx = co_b[:DXV].reshape(1, DXV)
  cb_rx = co_b[DXV:].reshape(1, RXV)

  pooled = []
  for h in range(_HALVES):
    lo = h * _BH
    dxi = dxseqs[lo:lo + _BH].reshape(-1).astype(jnp.int32)
    rxi = drugseqs[lo:lo + _BH].reshape(-1).astype(jnp.int32)
    pooled.append(_bag(dxi, rxi, EHRdxEmb, EHRdrugEmb))

  dps, rds, mts, parts = [], [], [], []
  for h in range(_HALVES):
    lo = h * _BH
    visit_bf, dp_h, rd_h, mt_h = _attn_call(
        _BH, pooled[h], attn_W, a_b, ac_w, ac_b, dp_W, dp_b2, r_w, r_b,
        m_w, m_b)
    pdx = _loss_call(_BH, visit_bf, co_Wb[:, :DXV], cb_dx,
                     dx_onehot[lo:lo + _BH], DXV)
    prx = _loss_call(_BH, visit_bf, co_Wb[:, DXV:], cb_rx,
                     drug_onehot[lo:lo + _BH], RXV)
    dps.append(dp_h)
    rds.append(rd_h)
    mts.append(mt_h)
    parts.extend(list(pdx) + list(prx))

  dpPred = jnp.concatenate(dps, axis=0)
  readPred = jnp.concatenate(rds, axis=0)
  mortPred = jnp.concatenate(mts, axis=0)
  co_loss = _combine_call(parts)[0, 0]
  return dpPred, readPred, mortPred, co_loss


# speedup vs baseline: 1.6888x; 1.6888x over previous
import functools

import jax
import jax.numpy as jnp
from jax import lax
from jax.experimental import pallas as pl
from jax.experimental.pallas import tpu as pltpu
from jax.experimental.pallas import tpu_sc as plsc

B, T, DXN, RXN = 128, 20, 40, 40
D = 256
DXV, RXV = 4880, 3000
ATTN = 128
DPL = 4880
SEG = B * T

NC, NS = 2, 16
NW = NC * NS
SEGW = SEG // NW


_NBUF = 4


def _make_bag(nseg, h):
  segw = nseg // NW
  nchunk = segw
  mesh = plsc.VectorSubcoreMesh(core_axis_name="c", subcore_axis_name="s")

  @functools.partial(
      pl.kernel,
      mesh=mesh,
      out_type=jax.ShapeDtypeStruct((nseg, D), jnp.float32),
      scratch_types=[
          pltpu.VMEM((segw * DXN,), jnp.int32),
          pltpu.VMEM((segw * RXN,), jnp.int32),
          pltpu.VMEM((_NBUF, DXN, D), jnp.float32),
          pltpu.VMEM((_NBUF, RXN, D), jnp.float32),
          pltpu.VMEM((segw, D), jnp.float32),
      ] + [pltpu.SemaphoreType.DMA] * (2 * _NBUF),
  )
  def bag(dxi_hbm, rxi_hbm, dxemb_hbm, rxemb_hbm, out_hbm,
          dxi_v, rxi_v, rows_dx, rows_rx, acc_v, *sems_flat):
    wid = lax.axis_index("s") * NC + lax.axis_index("c")
    base = h * nseg + wid * segw
    pltpu.sync_copy(dxi_hbm.at[pl.ds(base * DXN, segw * DXN)], dxi_v)
    pltpu.sync_copy(rxi_hbm.at[pl.ds(base * RXN, segw * RXN)], rxi_v)
    sems = tuple(
        (sems_flat[2 * u], sems_flat[2 * u + 1]) for u in range(_NBUF))

    def issue(c, slot):
      sd, sr = sems[slot]
      pltpu.async_copy(
          dxemb_hbm.at[dxi_v.at[pl.ds(c * DXN, DXN)]], rows_dx.at[slot], sd)
      pltpu.async_copy(
          rxemb_hbm.at[rxi_v.at[pl.ds(c * RXN, RXN)]], rows_rx.at[slot], sr)

    def consume(c, slot):
      def row_body(h, acc):
        r = 2 * h
        return tuple(
            acc[j]
            + (rows_dx[slot, r, pl.ds(16 * j, 16)]
               + rows_rx[slot, r, pl.ds(16 * j, 16)])
            + (rows_dx[slot, r + 1, pl.ds(16 * j, 16)]
               + rows_rx[slot, r + 1, pl.ds(16 * j, 16)])
            for j in range(D // 16))

      zeros = tuple(jnp.zeros((16,), jnp.float32) for _ in range(D // 16))
      acc = lax.fori_loop(0, DXN // 2, row_body, zeros)
      for j in range(D // 16):
        acc_v[c, pl.ds(16 * j, 16)] = acc[j]

    def drain(slot):
      sd, sr = sems[slot]
      pltpu.make_async_copy(
          dxemb_hbm.at[dxi_v.at[pl.ds(0, DXN)]], rows_dx.at[slot], sd).wait()
      pltpu.make_async_copy(
          rxemb_hbm.at[rxi_v.at[pl.ds(0, RXN)]], rows_rx.at[slot], sr).wait()

    for u in range(_NBUF - 1):
      issue(u, u)

    def ring_body(i, carry):
      c0 = _NBUF * i
      for u in range(_NBUF):
        c = c0 + u
        drain(u)
        consume(c, u)
        nxt = c + _NBUF - 1

        @pl.when(nxt < nchunk)
        def _():
          issue(nxt, (u + _NBUF - 1) % _NBUF)

      return carry

    lax.fori_loop(0, nchunk // _NBUF, ring_body, 0)
    pltpu.sync_copy(acc_v, out_hbm.at[pl.ds(wid * segw, segw)])

  return bag


_HALVES = 2
_BH = B // _HALVES
_SEGH = _BH * T
_bags = [_make_bag(_SEGH, h) for h in range(_HALVES)]


def _attn_body(nb, pooled_ref, attn_w_ref, attn_b_ref, attnc_w_ref,
               attnc_b_ref, dp_w_ref, dp_b_ref, read_w_ref, read_b_ref,
               mort_w_ref, mort_b_ref, visit_out, dp_out, read_out, mort_out):
  visit = jnp.tanh(pooled_ref[...])
  v3 = visit.reshape(nb, T, D)
  last = v3[:, T - 1, :]
  w1 = attn_w_ref[0:D, :]
  w2 = attn_w_ref[D:2 * D, :]
  h = jnp.dot(visit, w1, preferred_element_type=jnp.float32)
  h2 = jnp.dot(last, w2, preferred_element_type=jnp.float32)
  e = jnp.tanh(h.reshape(nb, T, ATTN) + h2[:, None, :] + attn_b_ref[...])
  sc = jnp.sum(e * attnc_w_ref[...][None, :, :], axis=-1) + attnc_b_ref[0, 0]
  m = jnp.max(sc, axis=1, keepdims=True)
  a = jnp.exp(sc - m)
  alpha = a / jnp.sum(a, axis=1, keepdims=True)
  pt = jnp.sum(alpha[:, :, None] * v3, axis=1)
  dp = jax.nn.sigmoid(
      jnp.dot(pt, dp_w_ref[...], preferred_element_type=jnp.float32)
      + dp_b_ref[...])
  rd = jax.nn.sigmoid(
      jnp.sum(pt * read_w_ref[...], axis=-1, keepdims=True) + read_b_ref[0, 0])
  mt = jax.nn.sigmoid(
      jnp.sum(pt * mort_w_ref[...], axis=-1, keepdims=True) + mort_b_ref[0, 0])
  visit_out[...] = visit.astype(jnp.bfloat16)
  dp_out[...] = dp
  read_out[...] = rd
  mort_out[...] = mt


def _attn_call(nb, pooled, attn_w, attn_b, attnc_w, attnc_b, dp_w, dp_b,
               read_w, read_b, mort_w, mort_b):
  nseg = nb * T
  return pl.pallas_call(
      functools.partial(_attn_body, nb),
      out_shape=(
          jax.ShapeDtypeStruct((nseg, D), jnp.bfloat16),
          jax.ShapeDtypeStruct((nb, DPL), jnp.float32),
          jax.ShapeDtypeStruct((nb, 1), jnp.float32),
          jax.ShapeDtypeStruct((nb, 1), jnp.float32),
      ),
  )(pooled, attn_w, attn_b, attnc_w, attnc_b, dp_w, dp_b, read_w, read_b,
    mort_w, mort_b)


_VT = 512


def _loss_body(nb, vocab, visit_ref, w_ref, b_ref, lab_ref,
               z_out, s1_out, sl_out):
  nseg = nb * T
  i = pl.program_id(0)

  @pl.when(i == 0)
  def _():
    z_out[...] = jnp.zeros_like(z_out)
    s1_out[...] = jnp.zeros_like(s1_out)
    sl_out[...] = jnp.zeros_like(sl_out)

  logits = (jnp.dot(visit_ref[...], w_ref[...],
                    preferred_element_type=jnp.float32) + b_ref[...])
  col = i * _VT + lax.broadcasted_iota(jnp.int32, (1, _VT), 1)
  valid = col < vocab
  lab = lab_ref[...].reshape(nseg, _VT)
  expl = jnp.where(valid, jnp.exp(logits), 0.0)
  s1 = jnp.where(valid, lab * logits, 0.0)
  sl = jnp.where(valid, lab, 0.0)
  z_out[...] += jnp.sum(expl, axis=1, keepdims=True)
  s1_out[...] += jnp.sum(s1, axis=1, keepdims=True)
  sl_out[...] += jnp.sum(sl, axis=1, keepdims=True)


def _loss_call(nb, h, visit_bf, w_bf, b2, labels, vocab):
  nseg = nb * T
  num_tiles = pl.cdiv(vocab, _VT)
  one = jax.ShapeDtypeStruct((nseg, 1), jnp.float32)
  return pl.pallas_call(
      functools.partial(_loss_body, nb, vocab),
      grid=(num_tiles,),
      in_specs=[
          pl.BlockSpec((nseg, D), lambda i: (0, 0)),
          pl.BlockSpec((D, _VT), lambda i: (0, i)),
          pl.BlockSpec((1, _VT), lambda i: (0, i)),
          pl.BlockSpec((nb, T, _VT), lambda i, _h=h: (_h, 0, i)),
      ],
      out_specs=(
          pl.BlockSpec((nseg, 1), lambda i: (0, 0)),
          pl.BlockSpec((nseg, 1), lambda i: (0, 0)),
          pl.BlockSpec((nseg, 1), lambda i: (0, 0)),
      ),
      out_shape=(one, one, one),
  )(visit_bf, w_bf, b2, labels)


def _combine_body(*refs):
  parts, out_ref = refs[:-1], refs[-1]
  total = None
  for h in range(_HALVES):
    z1, s11, sl1, z2, s12, sl2 = parts[6 * h:6 * h + 6]
    z = z1[...] + z2[...]
    s1 = s11[...] + s12[...]
    sl = sl1[...] + sl2[...]
    rows = s1 - jnp.log(z) * sl
    t = jnp.sum(rows, axis=0, keepdims=True)
    total = t if total is None else total + t
  out_ref[...] = -total / B


def _combine_call(parts):
  return pl.pallas_call(
      _combine_body,
      out_shape=jax.ShapeDtypeStruct((1, 1), jnp.float32),
  )(*parts)


def kernel(dxseqs, drugseqs, dx_onehot, drug_onehot, EHRdxEmb, EHRdrugEmb,
           attn_W, attn_b, attnC_W, attnC_b, dp_W, dp_b, read_W, read_b,
           mort_W, mort_b, co_W, co_b):
  co_Wb = co_W.astype(jnp.bfloat16)
  a_b = attn_b.reshape(1, ATTN)
  ac_w = attnC_W.reshape(1, ATTN)
  ac_b = attnC_b.reshape(1, 1)
  dp_b2 = dp_b.reshape(1, DPL)
  r_w = read_W.reshape(1, D)
  r_b = read_b.reshape(1, 1)
  m_w = mort_W.reshape(1, D)
  m_b = mort_b.reshape(1, 1)
  cb_dx = co_b[:DXV].reshape(1, DXV)
  cb_rx = co_b[DXV:].reshape(1, RXV)

  dxi = dxseqs.reshape(-1).astype(jnp.int32)
  rxi = drugseqs.reshape(-1).astype(jnp.int32)
  pooled = [_bags[h](dxi, rxi, EHRdxEmb, EHRdrugEmb)
            for h in range(_HALVES)]

  dps, rds, mts, parts = [], [], [], []
  for h in range(_HALVES):
    lo = h * _BH
    visit_bf, dp_h, rd_h, mt_h = _attn_call(
        _BH, pooled[h], attn_W, a_b, ac_w, ac_b, dp_W, dp_b2, r_w, r_b,
        m_w, m_b)
    pdx = _loss_call(_BH, h, visit_bf, co_Wb[:, :DXV], cb_dx,
                     dx_onehot, DXV)
    prx = _loss_call(_BH, h, visit_bf, co_Wb[:, DXV:], cb_rx,
                     drug_onehot, RXV)
    dps.append(dp_h)
    rds.append(rd_h)
    mts.append(mt_h)
    parts.extend(list(pdx) + list(prx))

  dpPred = jnp.concatenate(dps, axis=0)
  readPred = jnp.concatenate(rds, axis=0)
  mortPred = jnp.concatenate(mts, axis=0)
  co_loss = _combine_call(parts)[0, 0]
  return dpPred, readPred, mortPred, co_loss


# speedup vs baseline: 1.7727x; 1.0497x over previous
import functools

import jax
import jax.numpy as jnp
from jax import lax
from jax.experimental import pallas as pl
from jax.experimental.pallas import tpu as pltpu
from jax.experimental.pallas import tpu_sc as plsc

B, T, DXN, RXN = 128, 20, 40, 40
D = 256
DXV, RXV = 4880, 3000
ATTN = 128
DPL = 4880
SEG = B * T

NC, NS = 2, 16
NW = NC * NS
SEGW = SEG // NW


_NBUF = 4


def _make_bag(nseg, h):
  segw = nseg // NW
  nchunk = segw
  mesh = plsc.VectorSubcoreMesh(core_axis_name="c", subcore_axis_name="s")

  @functools.partial(
      pl.kernel,
      mesh=mesh,
      out_type=jax.ShapeDtypeStruct((nseg, D), jnp.float32),
      scratch_types=[
          pltpu.VMEM((segw * DXN,), jnp.int32),
          pltpu.VMEM((segw * RXN,), jnp.int32),
          pltpu.VMEM((_NBUF, DXN, D), jnp.float32),
          pltpu.VMEM((_NBUF, RXN, D), jnp.float32),
          pltpu.VMEM((segw, D), jnp.float32),
      ] + [pltpu.SemaphoreType.DMA] * (2 * _NBUF),
  )
  def bag(dxi_hbm, rxi_hbm, dxemb_hbm, rxemb_hbm, out_hbm,
          dxi_v, rxi_v, rows_dx, rows_rx, acc_v, *sems_flat):
    wid = lax.axis_index("s") * NC + lax.axis_index("c")
    base = h * nseg + wid * segw
    pltpu.sync_copy(dxi_hbm.at[pl.ds(base * DXN, segw * DXN)], dxi_v)
    pltpu.sync_copy(rxi_hbm.at[pl.ds(base * RXN, segw * RXN)], rxi_v)
    sems = tuple(
        (sems_flat[2 * u], sems_flat[2 * u + 1]) for u in range(_NBUF))

    def issue(c, slot):
      sd, sr = sems[slot]
      pltpu.async_copy(
          dxemb_hbm.at[dxi_v.at[pl.ds(c * DXN, DXN)]], rows_dx.at[slot], sd)
      pltpu.async_copy(
          rxemb_hbm.at[rxi_v.at[pl.ds(c * RXN, RXN)]], rows_rx.at[slot], sr)

    def consume(c, slot):
      def row_body(h, acc):
        r = 2 * h
        return tuple(
            acc[j]
            + (rows_dx[slot, r, pl.ds(16 * j, 16)]
               + rows_rx[slot, r, pl.ds(16 * j, 16)])
            + (rows_dx[slot, r + 1, pl.ds(16 * j, 16)]
               + rows_rx[slot, r + 1, pl.ds(16 * j, 16)])
            for j in range(D // 16))

      zeros = tuple(jnp.zeros((16,), jnp.float32) for _ in range(D // 16))
      acc = lax.fori_loop(0, DXN // 2, row_body, zeros)
      for j in range(D // 16):
        acc_v[c, pl.ds(16 * j, 16)] = acc[j]

    def drain(slot):
      sd, sr = sems[slot]
      pltpu.make_async_copy(
          dxemb_hbm.at[dxi_v.at[pl.ds(0, DXN)]], rows_dx.at[slot], sd).wait()
      pltpu.make_async_copy(
          rxemb_hbm.at[rxi_v.at[pl.ds(0, RXN)]], rows_rx.at[slot], sr).wait()

    for u in range(_NBUF - 1):
      issue(u, u)

    def ring_body(i, carry):
      c0 = _NBUF * i
      for u in range(_NBUF):
        c = c0 + u
        drain(u)
        consume(c, u)
        nxt = c + _NBUF - 1

        @pl.when(nxt < nchunk)
        def _():
          issue(nxt, (u + _NBUF - 1) % _NBUF)

      return carry

    lax.fori_loop(0, nchunk // _NBUF, ring_body, 0)
    pltpu.sync_copy(acc_v, out_hbm.at[pl.ds(wid * segw, segw)])

  return bag


_HALVES = 1
_BH = B // _HALVES
_SEGH = _BH * T
_bags = [_make_bag(_SEGH, h) for h in range(_HALVES)]


def _attn_body(nb, pooled_ref, attn_w_ref, attn_b_ref, attnc_w_ref,
               attnc_b_ref, dp_w_ref, dp_b_ref, read_w_ref, read_b_ref,
               mort_w_ref, mort_b_ref, visit_out, dp_out, read_out, mort_out):
  visit = jnp.tanh(pooled_ref[...])
  v3 = visit.reshape(nb, T, D)
  last = v3[:, T - 1, :]
  w1 = attn_w_ref[0:D, :]
  w2 = attn_w_ref[D:2 * D, :]
  h = jnp.dot(visit, w1, preferred_element_type=jnp.float32)
  h2 = jnp.dot(last, w2, preferred_element_type=jnp.float32)
  e = jnp.tanh(h.reshape(nb, T, ATTN) + h2[:, None, :] + attn_b_ref[...])
  sc = jnp.sum(e * attnc_w_ref[...][None, :, :], axis=-1) + attnc_b_ref[0, 0]
  m = jnp.max(sc, axis=1, keepdims=True)
  a = jnp.exp(sc - m)
  alpha = a / jnp.sum(a, axis=1, keepdims=True)
  pt = jnp.sum(alpha[:, :, None] * v3, axis=1)
  dp = jax.nn.sigmoid(
      jnp.dot(pt, dp_w_ref[...], preferred_element_type=jnp.float32)
      + dp_b_ref[...])
  rd = jax.nn.sigmoid(
      jnp.sum(pt * read_w_ref[...], axis=-1, keepdims=True) + read_b_ref[0, 0])
  mt = jax.nn.sigmoid(
      jnp.sum(pt * mort_w_ref[...], axis=-1, keepdims=True) + mort_b_ref[0, 0])
  visit_out[...] = visit.astype(jnp.bfloat16)
  dp_out[...] = dp
  read_out[...] = rd
  mort_out[...] = mt


def _attn_call(nb, pooled, attn_w, attn_b, attnc_w, attnc_b, dp_w, dp_b,
               read_w, read_b, mort_w, mort_b):
  nseg = nb * T
  return pl.pallas_call(
      functools.partial(_attn_body, nb),
      out_shape=(
          jax.ShapeDtypeStruct((nseg, D), jnp.bfloat16),
          jax.ShapeDtypeStruct((nb, DPL), jnp.float32),
          jax.ShapeDtypeStruct((nb, 1), jnp.float32),
          jax.ShapeDtypeStruct((nb, 1), jnp.float32),
      ),
  )(pooled, attn_w, attn_b, attnc_w, attnc_b, dp_w, dp_b, read_w, read_b,
    mort_w, mort_b)


_VT = 1024


def _loss_body(nb, vocab, visit_ref, w_ref, b_ref, lab_ref,
               z_out, s1_out, sl_out):
  nseg = nb * T
  i = pl.program_id(0)

  @pl.when(i == 0)
  def _():
    z_out[...] = jnp.zeros_like(z_out)
    s1_out[...] = jnp.zeros_like(s1_out)
    sl_out[...] = jnp.zeros_like(sl_out)

  logits = (jnp.dot(visit_ref[...], w_ref[...],
                    preferred_element_type=jnp.float32) + b_ref[...])
  col = i * _VT + lax.broadcasted_iota(jnp.int32, (1, _VT), 1)
  valid = col < vocab
  lab = lab_ref[...].reshape(nseg, _VT)
  expl = jnp.where(valid, jnp.exp(logits), 0.0)
  s1 = jnp.where(valid, lab * logits, 0.0)
  sl = jnp.where(valid, lab, 0.0)
  z_out[...] += jnp.sum(expl, axis=1, keepdims=True)
  s1_out[...] += jnp.sum(s1, axis=1, keepdims=True)
  sl_out[...] += jnp.sum(sl, axis=1, keepdims=True)


def _loss_call(nb, h, visit_bf, w_bf, b2, labels, vocab):
  nseg = nb * T
  num_tiles = pl.cdiv(vocab, _VT)
  one = jax.ShapeDtypeStruct((nseg, 1), jnp.float32)
  return pl.pallas_call(
      functools.partial(_loss_body, nb, vocab),
      grid=(num_tiles,),
      in_specs=[
          pl.BlockSpec((nseg, D), lambda i: (0, 0)),
          pl.BlockSpec((D, _VT), lambda i: (0, i)),
          pl.BlockSpec((1, _VT), lambda i: (0, i)),
          pl.BlockSpec((nb, T, _VT), lambda i, _h=h: (_h, 0, i)),
      ],
      out_specs=(
          pl.BlockSpec((nseg, 1), lambda i: (0, 0)),
          pl.BlockSpec((nseg, 1), lambda i: (0, 0)),
          pl.BlockSpec((nseg, 1), lambda i: (0, 0)),
      ),
      out_shape=(one, one, one),
  )(visit_bf, w_bf, b2, labels)


def _combine_body(*refs):
  parts, out_ref = refs[:-1], refs[-1]
  total = None
  for h in range(_HALVES):
    z1, s11, sl1, z2, s12, sl2 = parts[6 * h:6 * h + 6]
    z = z1[...] + z2[...]
    s1 = s11[...] + s12[...]
    sl = sl1[...] + sl2[...]
    rows = s1 - jnp.log(z) * sl
    t = jnp.sum(rows, axis=0, keepdims=True)
    total = t if total is None else total + t
  out_ref[...] = -total / B


def _combine_call(parts):
  return pl.pallas_call(
      _combine_body,
      out_shape=jax.ShapeDtypeStruct((1, 1), jnp.float32),
  )(*parts)


def kernel(dxseqs, drugseqs, dx_onehot, drug_onehot, EHRdxEmb, EHRdrugEmb,
           attn_W, attn_b, attnC_W, attnC_b, dp_W, dp_b, read_W, read_b,
           mort_W, mort_b, co_W, co_b):
  co_Wb = co_W.astype(jnp.bfloat16)
  a_b = attn_b.reshape(1, ATTN)
  ac_w = attnC_W.reshape(1, ATTN)
  ac_b = attnC_b.reshape(1, 1)
  dp_b2 = dp_b.reshape(1, DPL)
  r_w = read_W.reshape(1, D)
  r_b = read_b.reshape(1, 1)
  m_w = mort_W.reshape(1, D)
  m_b = mort_b.reshape(1, 1)
  cb_dx = co_b[:DXV].reshape(1, DXV)
  cb_rx = co_b[DXV:].reshape(1, RXV)

  dxi = dxseqs.reshape(-1).astype(jnp.int32)
  rxi = drugseqs.reshape(-1).astype(jnp.int32)
  pooled = [_bags[h](dxi, rxi, EHRdxEmb, EHRdrugEmb)
            for h in range(_HALVES)]

  dps, rds, mts, parts = [], [], [], []
  for h in range(_HALVES):
    lo = h * _BH
    visit_bf, dp_h, rd_h, mt_h = _attn_call(
        _BH, pooled[h], attn_W, a_b, ac_w, ac_b, dp_W, dp_b2, r_w, r_b,
        m_w, m_b)
    pdx = _loss_call(_BH, h, visit_bf, co_Wb[:, :DXV], cb_dx,
                     dx_onehot, DXV)
    prx = _loss_call(_BH, h, visit_bf, co_Wb[:, DXV:], cb_rx,
                     drug_onehot, RXV)
    dps.append(dp_h)
    rds.append(rd_h)
    mts.append(mt_h)
    parts.extend(list(pdx) + list(prx))

  dpPred = jnp.concatenate(dps, axis=0)
  readPred = jnp.concatenate(rds, axis=0)
  mortPred = jnp.concatenate(mts, axis=0)
  co_loss = _combine_call(parts)[0, 0]
  return dpPred, readPred, mortPred, co_loss


# speedup vs baseline: 1.8477x; 1.0423x over previous
import functools

import jax
import jax.numpy as jnp
from jax import lax
from jax.experimental import pallas as pl
from jax.experimental.pallas import tpu as pltpu
from jax.experimental.pallas import tpu_sc as plsc

B, T, DXN, RXN = 128, 20, 40, 40
D = 256
DXV, RXV = 4880, 3000
ATTN = 128
DPL = 4880
SEG = B * T

NC, NS = 2, 16
NW = NC * NS
SEGW = SEG // NW


_NBUF = 4


def _make_bag(nseg, h):
  segw = nseg // NW
  nchunk = segw
  mesh = plsc.VectorSubcoreMesh(core_axis_name="c", subcore_axis_name="s")

  @functools.partial(
      pl.kernel,
      mesh=mesh,
      out_type=jax.ShapeDtypeStruct((nseg, D), jnp.float32),
      scratch_types=[
          pltpu.VMEM((segw * DXN,), jnp.int32),
          pltpu.VMEM((segw * RXN,), jnp.int32),
          pltpu.VMEM((_NBUF, DXN, D // 2), jnp.int32),
          pltpu.VMEM((_NBUF, RXN, D // 2), jnp.int32),
          pltpu.VMEM((segw, D), jnp.float32),
      ] + [pltpu.SemaphoreType.DMA] * (2 * _NBUF),
  )
  def bag(dxi_hbm, rxi_hbm, dxemb_hbm, rxemb_hbm, out_hbm,
          dxi_v, rxi_v, rows_dx, rows_rx, acc_v, *sems_flat):
    wid = lax.axis_index("s") * NC + lax.axis_index("c")
    base = h * nseg + wid * segw
    pltpu.sync_copy(dxi_hbm.at[pl.ds(base * DXN, segw * DXN)], dxi_v)
    pltpu.sync_copy(rxi_hbm.at[pl.ds(base * RXN, segw * RXN)], rxi_v)
    sems = tuple(
        (sems_flat[2 * u], sems_flat[2 * u + 1]) for u in range(_NBUF))

    def issue(c, slot):
      sd, sr = sems[slot]
      pltpu.async_copy(
          dxemb_hbm.at[dxi_v.at[pl.ds(c * DXN, DXN)]], rows_dx.at[slot], sd)
      pltpu.async_copy(
          rxemb_hbm.at[rxi_v.at[pl.ds(c * RXN, RXN)]], rows_rx.at[slot], sr)

    def consume(c, slot):
      def row_body(r, acc):
        new = []
        for j in range(D // 32):
          wdx = rows_dx[slot, r, pl.ds(16 * j, 16)]
          wrx = rows_rx[slot, r, pl.ds(16 * j, 16)]
          lo = (acc[2 * j]
                + lax.bitcast_convert_type(wdx << 16, jnp.float32)
                + lax.bitcast_convert_type(wrx << 16, jnp.float32))
          hi = (acc[2 * j + 1]
                + lax.bitcast_convert_type(wdx, jnp.float32)
                + lax.bitcast_convert_type(wrx, jnp.float32))
          new += [lo, hi]
        return tuple(new)

      zeros = tuple(jnp.zeros((16,), jnp.float32) for _ in range(D // 16))
      acc = lax.fori_loop(0, DXN, row_body, zeros)
      for j in range(D // 32):
        acc_v[c, pl.ds(32 * j, 16)] = acc[2 * j]
        acc_v[c, pl.ds(32 * j + 16, 16)] = acc[2 * j + 1]

    def drain(slot):
      sd, sr = sems[slot]
      pltpu.make_async_copy(
          dxemb_hbm.at[dxi_v.at[pl.ds(0, DXN)]], rows_dx.at[slot], sd).wait()
      pltpu.make_async_copy(
          rxemb_hbm.at[rxi_v.at[pl.ds(0, RXN)]], rows_rx.at[slot], sr).wait()

    for u in range(_NBUF - 1):
      issue(u, u)

    def ring_body(i, carry):
      c0 = _NBUF * i
      for u in range(_NBUF):
        c = c0 + u
        drain(u)
        consume(c, u)
        nxt = c + _NBUF - 1

        @pl.when(nxt < nchunk)
        def _():
          issue(nxt, (u + _NBUF - 1) % _NBUF)

      return carry

    lax.fori_loop(0, nchunk // _NBUF, ring_body, 0)
    pltpu.sync_copy(acc_v, out_hbm.at[pl.ds(wid * segw, segw)])

  return bag


_HALVES = 1
_BH = B // _HALVES
_SEGH = _BH * T
_bags = [_make_bag(_SEGH, h) for h in range(_HALVES)]


def _attn_body(nb, pooled_ref, attn_w_ref, attn_b_ref, attnc_w_ref,
               attnc_b_ref, dp_w_ref, dp_b_ref, read_w_ref, read_b_ref,
               mort_w_ref, mort_b_ref, visit_out, dp_out, read_out, mort_out):
  visit = jnp.tanh(pooled_ref[...])
  v3 = visit.reshape(nb, T, D)
  last = v3[:, T - 1, :]
  w1 = attn_w_ref[0:D, :]
  w2 = attn_w_ref[D:2 * D, :]
  h = jnp.dot(visit, w1, preferred_element_type=jnp.float32)
  h2 = jnp.dot(last, w2, preferred_element_type=jnp.float32)
  e = jnp.tanh(h.reshape(nb, T, ATTN) + h2[:, None, :] + attn_b_ref[...])
  sc = jnp.sum(e * attnc_w_ref[...][None, :, :], axis=-1) + attnc_b_ref[0, 0]
  m = jnp.max(sc, axis=1, keepdims=True)
  a = jnp.exp(sc - m)
  alpha = a / jnp.sum(a, axis=1, keepdims=True)
  pt = jnp.sum(alpha[:, :, None] * v3, axis=1)
  dp = jax.nn.sigmoid(
      jnp.dot(pt, dp_w_ref[...], preferred_element_type=jnp.float32)
      + dp_b_ref[...])
  rd = jax.nn.sigmoid(
      jnp.sum(pt * read_w_ref[...], axis=-1, keepdims=True) + read_b_ref[0, 0])
  mt = jax.nn.sigmoid(
      jnp.sum(pt * mort_w_ref[...], axis=-1, keepdims=True) + mort_b_ref[0, 0])
  visit_out[...] = visit.astype(jnp.bfloat16)
  dp_out[...] = dp
  read_out[...] = rd
  mort_out[...] = mt


def _attn_call(nb, pooled, attn_w, attn_b, attnc_w, attnc_b, dp_w, dp_b,
               read_w, read_b, mort_w, mort_b):
  nseg = nb * T
  return pl.pallas_call(
      functools.partial(_attn_body, nb),
      out_shape=(
          jax.ShapeDtypeStruct((nseg, D), jnp.bfloat16),
          jax.ShapeDtypeStruct((nb, DPL), jnp.float32),
          jax.ShapeDtypeStruct((nb, 1), jnp.float32),
          jax.ShapeDtypeStruct((nb, 1), jnp.float32),
      ),
  )(pooled, attn_w, attn_b, attnc_w, attnc_b, dp_w, dp_b, read_w, read_b,
    mort_w, mort_b)


_VT = 512


def _loss_body(nb, vocab, visit_ref, w_ref, b_ref, lab_ref,
               z_out, s1_out, sl_out):
  nseg = nb * T
  i = pl.program_id(0)

  @pl.when(i == 0)
  def _():
    z_out[...] = jnp.zeros_like(z_out)
    s1_out[...] = jnp.zeros_like(s1_out)
    sl_out[...] = jnp.zeros_like(sl_out)

  logits = (jnp.dot(visit_ref[...], w_ref[...],
                    preferred_element_type=jnp.float32) + b_ref[...])
  col = i * _VT + lax.broadcasted_iota(jnp.int32, (1, _VT), 1)
  valid = col < vocab
  lab = lab_ref[...].reshape(nseg, _VT)
  expl = jnp.where(valid, jnp.exp(logits), 0.0)
  s1 = jnp.where(valid, lab * logits, 0.0)
  sl = jnp.where(valid, lab, 0.0)
  z_out[...] += jnp.sum(expl, axis=1, keepdims=True)
  s1_out[...] += jnp.sum(s1, axis=1, keepdims=True)
  sl_out[...] += jnp.sum(sl, axis=1, keepdims=True)


def _loss_call(nb, h, visit_bf, w_bf, b2, labels, vocab):
  nseg = nb * T
  num_tiles = pl.cdiv(vocab, _VT)
  one = jax.ShapeDtypeStruct((nseg, 1), jnp.float32)
  return pl.pallas_call(
      functools.partial(_loss_body, nb, vocab),
      grid=(num_tiles,),
      in_specs=[
          pl.BlockSpec((nseg, D), lambda i: (0, 0)),
          pl.BlockSpec((D, _VT), lambda i: (0, i)),
          pl.BlockSpec((1, _VT), lambda i: (0, i)),
          pl.BlockSpec((nb, T, _VT), lambda i, _h=h: (_h, 0, i)),
      ],
      out_specs=(
          pl.BlockSpec((nseg, 1), lambda i: (0, 0)),
          pl.BlockSpec((nseg, 1), lambda i: (0, 0)),
          pl.BlockSpec((nseg, 1), lambda i: (0, 0)),
      ),
      out_shape=(one, one, one),
  )(visit_bf, w_bf, b2, labels)


def _combine_body(*refs):
  parts, out_ref = refs[:-1], refs[-1]
  total = None
  for h in range(_HALVES):
    z1, s11, sl1, z2, s12, sl2 = parts[6 * h:6 * h + 6]
    z = z1[...] + z2[...]
    s1 = s11[...] + s12[...]
    sl = sl1[...] + sl2[...]
    rows = s1 - jnp.log(z) * sl
    t = jnp.sum(rows, axis=0, keepdims=True)
    total = t if total is None else total + t
  out_ref[...] = -total / B


def _combine_call(parts):
  return pl.pallas_call(
      _combine_body,
      out_shape=jax.ShapeDtypeStruct((1, 1), jnp.float32),
  )(*parts)


def kernel(dxseqs, drugseqs, dx_onehot, drug_onehot, EHRdxEmb, EHRdrugEmb,
           attn_W, attn_b, attnC_W, attnC_b, dp_W, dp_b, read_W, read_b,
           mort_W, mort_b, co_W, co_b):
  co_Wb = co_W.astype(jnp.bfloat16)
  a_b = attn_b.reshape(1, ATTN)
  ac_w = attnC_W.reshape(1, ATTN)
  ac_b = attnC_b.reshape(1, 1)
  dp_b2 = dp_b.reshape(1, DPL)
  r_w = read_W.reshape(1, D)
  r_b = read_b.reshape(1, 1)
  m_w = mort_W.reshape(1, D)
  m_b = mort_b.reshape(1, 1)
  cb_dx = co_b[:DXV].reshape(1, DXV)
  cb_rx = co_b[DXV:].reshape(1, RXV)

  dxi = dxseqs.reshape(-1).astype(jnp.int32)
  rxi = drugseqs.reshape(-1).astype(jnp.int32)

  def _pack(tbl):
    tb = tbl.astype(jnp.bfloat16).reshape(-1, D // 32, 2, 16)
    tb = jnp.swapaxes(tb, 2, 3)
    return lax.bitcast_convert_type(tb, jnp.int32).reshape(-1, D // 2)

  pooled = [_bags[h](dxi, rxi, _pack(EHRdxEmb), _pack(EHRdrugEmb))
            for h in range(_HALVES)]

  dps, rds, mts, parts = [], [], [], []
  for h in range(_HALVES):
    lo = h * _BH
    visit_bf, dp_h, rd_h, mt_h = _attn_call(
        _BH, pooled[h], attn_W, a_b, ac_w, ac_b, dp_W, dp_b2, r_w, r_b,
        m_w, m_b)
    pdx = _loss_call(_BH, h, visit_bf, co_Wb[:, :DXV], cb_dx,
                     dx_onehot, DXV)
    prx = _loss_call(_BH, h, visit_bf, co_Wb[:, DXV:], cb_rx,
                     drug_onehot, RXV)
    dps.append(dp_h)
    rds.append(rd_h)
    mts.append(mt_h)
    parts.extend(list(pdx) + list(prx))

  dpPred = jnp.concatenate(dps, axis=0)
  readPred = jnp.concatenate(rds, axis=0)
  mortPred = jnp.concatenate(mts, axis=0)
  co_loss = _combine_call(parts)[0, 0]
  return dpPred, readPred, mortPred, co_loss


# speedup vs baseline: 1.8822x; 1.0186x over previous
import functools

import jax
import jax.numpy as jnp
from jax import lax
from jax.experimental import pallas as pl
from jax.experimental.pallas import tpu as pltpu
from jax.experimental.pallas import tpu_sc as plsc

B, T, DXN, RXN = 128, 20, 40, 40
D = 256
DXV, RXV = 4880, 3000
ATTN = 128
DPL = 4880
SEG = B * T

NC, NS = 2, 16
NW = NC * NS
SEGW = SEG // NW


_NBUF = 4


def _make_bag(nseg, h):
  segw = nseg // NW
  nchunk = segw
  mesh = plsc.VectorSubcoreMesh(core_axis_name="c", subcore_axis_name="s")

  @functools.partial(
      pl.kernel,
      mesh=mesh,
      out_type=jax.ShapeDtypeStruct((nseg, D), jnp.float32),
      scratch_types=[
          pltpu.VMEM((segw * DXN,), jnp.int32),
          pltpu.VMEM((segw * RXN,), jnp.int32),
          pltpu.VMEM((_NBUF, DXN, D // 2), jnp.int32),
          pltpu.VMEM((_NBUF, RXN, D // 2), jnp.int32),
          pltpu.VMEM((segw, D), jnp.float32),
      ] + [pltpu.SemaphoreType.DMA] * (2 * _NBUF),
  )
  def bag(dxi_hbm, rxi_hbm, dxemb_hbm, rxemb_hbm, out_hbm,
          dxi_v, rxi_v, rows_dx, rows_rx, acc_v, *sems_flat):
    wid = lax.axis_index("s") * NC + lax.axis_index("c")
    base = h * nseg + wid * segw
    pltpu.sync_copy(dxi_hbm.at[pl.ds(base * DXN, segw * DXN)], dxi_v)
    pltpu.sync_copy(rxi_hbm.at[pl.ds(base * RXN, segw * RXN)], rxi_v)
    sems = tuple(
        (sems_flat[2 * u], sems_flat[2 * u + 1]) for u in range(_NBUF))

    def issue(c, slot):
      sd, sr = sems[slot]
      pltpu.async_copy(
          dxemb_hbm.at[dxi_v.at[pl.ds(c * DXN, DXN)]], rows_dx.at[slot], sd)
      pltpu.async_copy(
          rxemb_hbm.at[rxi_v.at[pl.ds(c * RXN, RXN)]], rows_rx.at[slot], sr)

    def consume(c, slot):
      def row_body(r, acc):
        new = []
        for j in range(D // 32):
          wdx = rows_dx[slot, r, pl.ds(16 * j, 16)]
          wrx = rows_rx[slot, r, pl.ds(16 * j, 16)]
          lo = (acc[2 * j]
                + lax.bitcast_convert_type(wdx << 16, jnp.float32)
                + lax.bitcast_convert_type(wrx << 16, jnp.float32))
          hi = (acc[2 * j + 1]
                + lax.bitcast_convert_type(wdx, jnp.float32)
                + lax.bitcast_convert_type(wrx, jnp.float32))
          new += [lo, hi]
        return tuple(new)

      zeros = tuple(jnp.zeros((16,), jnp.float32) for _ in range(D // 16))
      acc = lax.fori_loop(0, DXN, row_body, zeros)
      for j in range(D // 32):
        acc_v[c, pl.ds(32 * j, 16)] = acc[2 * j]
        acc_v[c, pl.ds(32 * j + 16, 16)] = acc[2 * j + 1]

    def drain(slot):
      sd, sr = sems[slot]
      pltpu.make_async_copy(
          dxemb_hbm.at[dxi_v.at[pl.ds(0, DXN)]], rows_dx.at[slot], sd).wait()
      pltpu.make_async_copy(
          rxemb_hbm.at[rxi_v.at[pl.ds(0, RXN)]], rows_rx.at[slot], sr).wait()

    for u in range(_NBUF - 1):
      issue(u, u)

    def ring_body(i, carry):
      c0 = _NBUF * i
      for u in range(_NBUF):
        c = c0 + u
        drain(u)
        consume(c, u)
        nxt = c + _NBUF - 1

        @pl.when(nxt < nchunk)
        def _():
          issue(nxt, (u + _NBUF - 1) % _NBUF)

      return carry

    lax.fori_loop(0, nchunk // _NBUF, ring_body, 0)
    pltpu.sync_copy(acc_v, out_hbm.at[pl.ds(wid * segw, segw)])

  return bag


_HALVES = 1
_BH = B // _HALVES
_SEGH = _BH * T
_bags = [_make_bag(_SEGH, h) for h in range(_HALVES)]


import numpy as np

_PLO = np.zeros((D, D // 2), np.float32)
_PHI = np.zeros((D, D // 2), np.float32)
for _k in range(D // 2):
  _PLO[32 * (_k // 16) + (_k % 16), _k] = 1.0
  _PHI[32 * (_k // 16) + 16 + (_k % 16), _k] = 1.0


def _pack_body(tbl_ref, plo_ref, phi_ref, out_ref):
  x = tbl_ref[...]
  lo_f = jnp.dot(x, plo_ref[...], preferred_element_type=jnp.float32)
  hi_f = jnp.dot(x, phi_ref[...], preferred_element_type=jnp.float32)
  il = lax.bitcast_convert_type(lo_f, jnp.int32)
  ih = lax.bitcast_convert_type(hi_f, jnp.int32)
  lo_w = lax.shift_right_logical(il + 0x8000, 16)
  hi_w = (ih + 0x8000) & jnp.int32(-65536)
  out_ref[...] = hi_w | lo_w


def _pack(tbl):
  v = tbl.shape[0]
  return pl.pallas_call(
      _pack_body,
      out_shape=jax.ShapeDtypeStruct((v, D // 2), jnp.int32),
  )(tbl, jnp.asarray(_PLO), jnp.asarray(_PHI))


def _attn_body(nb, pooled_ref, attn_w_ref, attn_b_ref, attnc_w_ref,
               attnc_b_ref, dp_w_ref, dp_b_ref, read_w_ref, read_b_ref,
               mort_w_ref, mort_b_ref, visit_out, dp_out, read_out, mort_out):
  visit = jnp.tanh(pooled_ref[...])
  v3 = visit.reshape(nb, T, D)
  last = v3[:, T - 1, :]
  w1 = attn_w_ref[0:D, :]
  w2 = attn_w_ref[D:2 * D, :]
  h = jnp.dot(visit, w1, preferred_element_type=jnp.float32)
  h2 = jnp.dot(last, w2, preferred_element_type=jnp.float32)
  e = jnp.tanh(h.reshape(nb, T, ATTN) + h2[:, None, :] + attn_b_ref[...])
  sc = jnp.sum(e * attnc_w_ref[...][None, :, :], axis=-1) + attnc_b_ref[0, 0]
  m = jnp.max(sc, axis=1, keepdims=True)
  a = jnp.exp(sc - m)
  alpha = a / jnp.sum(a, axis=1, keepdims=True)
  pt = jnp.sum(alpha[:, :, None] * v3, axis=1)
  dp = jax.nn.sigmoid(
      jnp.dot(pt, dp_w_ref[...], preferred_element_type=jnp.float32)
      + dp_b_ref[...])
  rd = jax.nn.sigmoid(
      jnp.sum(pt * read_w_ref[...], axis=-1, keepdims=True) + read_b_ref[0, 0])
  mt = jax.nn.sigmoid(
      jnp.sum(pt * mort_w_ref[...], axis=-1, keepdims=True) + mort_b_ref[0, 0])
  visit_out[...] = visit.astype(jnp.bfloat16)
  dp_out[...] = dp
  read_out[...] = rd
  mort_out[...] = mt


def _attn_call(nb, pooled, attn_w, attn_b, attnc_w, attnc_b, dp_w, dp_b,
               read_w, read_b, mort_w, mort_b):
  nseg = nb * T
  return pl.pallas_call(
      functools.partial(_attn_body, nb),
      out_shape=(
          jax.ShapeDtypeStruct((nseg, D), jnp.bfloat16),
          jax.ShapeDtypeStruct((nb, DPL), jnp.float32),
          jax.ShapeDtypeStruct((nb, 1), jnp.float32),
          jax.ShapeDtypeStruct((nb, 1), jnp.float32),
      ),
  )(pooled, attn_w, attn_b, attnc_w, attnc_b, dp_w, dp_b, read_w, read_b,
    mort_w, mort_b)


_VT = 512


def _loss_body(nb, vocab, visit_ref, w_ref, b_ref, lab_ref,
               z_out, s1_out, sl_out):
  nseg = nb * T
  i = pl.program_id(0)

  @pl.when(i == 0)
  def _():
    z_out[...] = jnp.zeros_like(z_out)
    s1_out[...] = jnp.zeros_like(s1_out)
    sl_out[...] = jnp.zeros_like(sl_out)

  logits = (jnp.dot(visit_ref[...], w_ref[...],
                    preferred_element_type=jnp.float32) + b_ref[...])
  col = i * _VT + lax.broadcasted_iota(jnp.int32, (1, _VT), 1)
  valid = col < vocab
  lab = lab_ref[...].reshape(nseg, _VT)
  expl = jnp.where(valid, jnp.exp(logits), 0.0)
  s1 = jnp.where(valid, lab * logits, 0.0)
  sl = jnp.where(valid, lab, 0.0)
  z_out[...] += jnp.sum(expl, axis=1, keepdims=True)
  s1_out[...] += jnp.sum(s1, axis=1, keepdims=True)
  sl_out[...] += jnp.sum(sl, axis=1, keepdims=True)


def _loss_call(nb, h, visit_bf, w_bf, b2, labels, vocab):
  nseg = nb * T
  num_tiles = pl.cdiv(vocab, _VT)
  one = jax.ShapeDtypeStruct((nseg, 1), jnp.float32)
  return pl.pallas_call(
      functools.partial(_loss_body, nb, vocab),
      grid=(num_tiles,),
      in_specs=[
          pl.BlockSpec((nseg, D), lambda i: (0, 0)),
          pl.BlockSpec((D, _VT), lambda i: (0, i)),
          pl.BlockSpec((1, _VT), lambda i: (0, i)),
          pl.BlockSpec((nb, T, _VT), lambda i, _h=h: (_h, 0, i)),
      ],
      out_specs=(
          pl.BlockSpec((nseg, 1), lambda i: (0, 0)),
          pl.BlockSpec((nseg, 1), lambda i: (0, 0)),
          pl.BlockSpec((nseg, 1), lambda i: (0, 0)),
      ),
      out_shape=(one, one, one),
  )(visit_bf, w_bf, b2, labels)


def _combine_body(*refs):
  parts, out_ref = refs[:-1], refs[-1]
  total = None
  for h in range(_HALVES):
    z1, s11, sl1, z2, s12, sl2 = parts[6 * h:6 * h + 6]
    z = z1[...] + z2[...]
    s1 = s11[...] + s12[...]
    sl = sl1[...] + sl2[...]
    rows = s1 - jnp.log(z) * sl
    t = jnp.sum(rows, axis=0, keepdims=True)
    total = t if total is None else total + t
  out_ref[...] = -total / B


def _combine_call(parts):
  return pl.pallas_call(
      _combine_body,
      out_shape=jax.ShapeDtypeStruct((1, 1), jnp.float32),
  )(*parts)


def kernel(dxseqs, drugseqs, dx_onehot, drug_onehot, EHRdxEmb, EHRdrugEmb,
           attn_W, attn_b, attnC_W, attnC_b, dp_W, dp_b, read_W, read_b,
           mort_W, mort_b, co_W, co_b):
  co_Wb = co_W.astype(jnp.bfloat16)
  a_b = attn_b.reshape(1, ATTN)
  ac_w = attnC_W.reshape(1, ATTN)
  ac_b = attnC_b.reshape(1, 1)
  dp_b2 = dp_b.reshape(1, DPL)
  r_w = read_W.reshape(1, D)
  r_b = read_b.reshape(1, 1)
  m_w = mort_W.reshape(1, D)
  m_b = mort_b.reshape(1, 1)
  cb_dx = co_b[:DXV].reshape(1, DXV)
  cb_rx = co_b[DXV:].reshape(1, RXV)

  dxi = dxseqs.reshape(-1).astype(jnp.int32)
  rxi = drugseqs.reshape(-1).astype(jnp.int32)

  pooled = [_bags[h](dxi, rxi, _pack(EHRdxEmb), _pack(EHRdrugEmb))
            for h in range(_HALVES)]

  dps, rds, mts, parts = [], [], [], []
  for h in range(_HALVES):
    lo = h * _BH
    visit_bf, dp_h, rd_h, mt_h = _attn_call(
        _BH, pooled[h], attn_W, a_b, ac_w, ac_b, dp_W, dp_b2, r_w, r_b,
        m_w, m_b)
    pdx = _loss_call(_BH, h, visit_bf, co_Wb[:, :DXV], cb_dx,
                     dx_onehot, DXV)
    prx = _loss_call(_BH, h, visit_bf, co_Wb[:, DXV:], cb_rx,
                     drug_onehot, RXV)
    dps.append(dp_h)
    rds.append(rd_h)
    mts.append(mt_h)
    parts.extend(list(pdx) + list(prx))

  dpPred = jnp.concatenate(dps, axis=0)
  readPred = jnp.concatenate(rds, axis=0)
  mortPred = jnp.concatenate(mts, axis=0)
  co_loss = _combine_call(parts)[0, 0]
  return dpPred, readPred, mortPred, co_loss
